# baseline (speedup 1.0000x reference)
import functools

import jax
import jax.numpy as jnp
from jax import lax
from jax.experimental import pallas as pl
from jax.experimental.pallas import tpu as pltpu
from jax.experimental.pallas import tpu_sc as plsc

D_MODEL = 64
_INFO = plsc.get_sparse_core_info()
_NC, _NS = _INFO.num_cores, _INFO.num_subcores
_NW = _NC * _NS

_CHUNK = 512
_SUB = 128
_NSUB = _CHUNK // _SUB

_VB = 256
_V = 1000000
_NBLK = _V // _VB
_REM_V0 = _NBLK * _VB
_REM = _V - _REM_V0
_NB_W = 124


def _transpose_call():
    mesh = plsc.VectorSubcoreMesh(core_axis_name="c", subcore_axis_name="s")

    @functools.partial(
        pl.kernel,
        out_type=jax.ShapeDtypeStruct((_V // 2, 128), jnp.float32),
        mesh=mesh,
        scratch_types=[
            pltpu.VMEM((D_MODEL, _VB), jnp.float32),
            pltpu.VMEM((D_MODEL, _VB), jnp.float32),
            pltpu.VMEM((_VB // 2, 128), jnp.float32),
            pltpu.VMEM((_VB // 2, 128), jnp.float32),
            pltpu.VMEM((D_MODEL, _REM), jnp.float32),
            pltpu.SemaphoreType.DMA,
            pltpu.SemaphoreType.DMA,
            pltpu.SemaphoreType.DMA,
            pltpu.SemaphoreType.DMA,
        ],
        compiler_params=pltpu.CompilerParams(
            use_tc_tiling_on_sc=True, needs_layout_passes=False),
    )
    def k1(wt_hbm, tail_hbm, out_hbm, in0, in1, so0, so1, invr,
           if0, if1, of0, of1):
        wid = lax.axis_index("s") * _NC + lax.axis_index("c")
        base = wid * (_NBLK // _NW) + jnp.minimum(wid, _NBLK % _NW)
        bufs = ((in0, so0, if0, of0), (in1, so1, if1, of1))
        iota = lax.iota(jnp.int32, 16)

        def blk(i):
            return jnp.minimum(base + i, _NBLK - 1)

        for b in (0, 1):
            inv, _, isem, _ = bufs[b]
            pltpu.async_copy(wt_hbm.at[:, pl.ds(blk(b) * _VB, _VB)],
                             inv, isem)

        def pair(p, carry):
            for b in (0, 1):
                inv, sov, isem, osem = bufs[b]
                i = 2 * p + b
                j = blk(i)
                pltpu.make_async_copy(
                    wt_hbm.at[:, pl.ds(j * _VB, _VB)], inv, isem).wait()

                @pl.when(p > 0)
                def _():
                    pltpu.make_async_copy(
                        sov, out_hbm.at[pl.ds(blk(i - 2) * (_VB // 2),
                                              _VB // 2)], osem).wait()

                def drow(d, carry2):
                    for k in range(_VB // 16):
                        l = iota + (k * 16)
                        row = lax.shift_right_logical(l, 1)
                        col = lax.shift_left(lax.bitwise_and(l, 1), 6) + d
                        vals = inv[d, pl.ds(k * 16, 16)]
                        plsc.store_scatter(sov, [row, col], vals)
                    return carry2

                lax.fori_loop(0, D_MODEL, drow, 0)

                pltpu.async_copy(
                    wt_hbm.at[:, pl.ds(blk(i + 2) * _VB, _VB)], inv, isem)
                pltpu.async_copy(
                    sov, out_hbm.at[pl.ds(j * (_VB // 2), _VB // 2)], osem)
            return carry

        lax.fori_loop(0, _NB_W // 2, pair, 0)

        for b in (0, 1):
            inv, sov, isem, osem = bufs[b]
            i = _NB_W - 2 + b
            pltpu.make_async_copy(
                sov, out_hbm.at[pl.ds(blk(i) * (_VB // 2), _VB // 2)],
                osem).wait()
            pltpu.make_async_copy(
                wt_hbm.at[:, pl.ds(blk(i + 2) * _VB, _VB)], inv, isem).wait()

        @pl.when(wid == _NW - 1)
        def _():
            _, sov, _, _ = bufs[0]
            pltpu.sync_copy(tail_hbm, invr)

            def drow_r(d, carry2):
                for k in range(_REM // 16):
                    l = iota + (k * 16)
                    row = lax.shift_right_logical(l, 1)
                    col = lax.shift_left(lax.bitwise_and(l, 1), 6) + d
                    vals = invr[d, pl.ds(k * 16, 16)]
                    plsc.store_scatter(sov, [row, col], vals)
                return carry2

            lax.fori_loop(0, D_MODEL, drow_r, 0)
            pltpu.sync_copy(sov.at[pl.ds(0, _REM // 2)],
                            out_hbm.at[pl.ds(_REM_V0 // 2, _REM // 2)])

    return k1


def _gather_call(total_rows):
    n_per_w = total_rows // _NW
    n_chunks = n_per_w // _CHUNK
    n_pairs = n_chunks // 2
    mesh = plsc.VectorSubcoreMesh(core_axis_name="c", subcore_axis_name="s")

    @functools.partial(
        pl.kernel,
        out_type=jax.ShapeDtypeStruct((total_rows, 128), jnp.float32),
        mesh=mesh,
        scratch_types=[
            pltpu.VMEM((_NSUB, _SUB), jnp.int32),
            pltpu.VMEM((_NSUB, _SUB), jnp.int32),
            pltpu.VMEM((_CHUNK, D_MODEL), jnp.float32),
            pltpu.VMEM((_CHUNK, D_MODEL), jnp.float32),
            pltpu.SemaphoreType.DMA,
            pltpu.SemaphoreType.DMA,
            pltpu.SemaphoreType.DMA,
            pltpu.SemaphoreType.DMA,
            pltpu.SemaphoreType.DMA,
            pltpu.SemaphoreType.DMA,
        ],
        compiler_params=pltpu.CompilerParams(use_tc_tiling_on_sc=False),
    )
    def emb(w_hbm, x_hbm, out_hbm, idx0, idx1, rows0, rows1,
            isem0, isem1, gsem0, gsem1, osem0, osem1):
        wid = lax.axis_index("s") * _NC + lax.axis_index("c")
        base = wid * (n_per_w // _SUB)
        bufs = ((idx0, rows0, isem0, gsem0, osem0),
                (idx1, rows1, isem1, gsem1, osem1))

        for b in (0, 1):
            idxv, _, isem, _, _ = bufs[b]
            pltpu.async_copy(x_hbm.at[pl.ds(base + b * _NSUB, _NSUB)],
                             idxv, isem)

        def pair(p, carry):
            for b in (0, 1):
                idxv, rowsv, isem, gsem, osem = bufs[b]
                i = 2 * p + b
                g = base + i * _NSUB

                @pl.when(p > 0)
                def _():
                    pltpu.make_async_copy(
                        rowsv, out_hbm.at[pl.ds((g - 2 * _NSUB) * _SUB,
                                                _CHUNK),
                                          pl.ds(0, D_MODEL)], osem).wait()

                pltpu.make_async_copy(
                    x_hbm.at[pl.ds(g, _NSUB)], idxv, isem).wait()

                for j in range(_NSUB):
                    pltpu.async_copy(w_hbm.at[idxv.at[j]],
                                     rowsv.at[pl.ds(j * _SUB, _SUB)], gsem)
                for j in range(_NSUB):
                    pltpu.make_async_copy(
                        w_hbm.at[idxv.at[j]],
                        rowsv.at[pl.ds(j * _SUB, _SUB)], gsem).wait()

                g_next = base + lax.rem(i + 2, n_chunks) * _NSUB
                pltpu.async_copy(x_hbm.at[pl.ds(g_next, _NSUB)], idxv, isem)

                pltpu.async_copy(rowsv,
                                 out_hbm.at[pl.ds(g * _SUB, _CHUNK),
                                            pl.ds(0, D_MODEL)], osem)
            return carry

        lax.fori_loop(0, n_pairs, pair, 0)

        for b in (0, 1):
            idxv, rowsv, isem, _, osem = bufs[b]
            i = n_chunks - 2 + b
            g = base + i * _NSUB
            pltpu.make_async_copy(
                rowsv, out_hbm.at[pl.ds(g * _SUB, _CHUNK),
                                  pl.ds(0, D_MODEL)], osem).wait()
            pltpu.make_async_copy(
                x_hbm.at[pl.ds(base + b * _NSUB, _NSUB)], idxv, isem).wait()

    return emb


def kernel(x, weight):
    b, s = x.shape
    total = b * s
    w_tail = weight[_REM_V0:, :].T
    w_packed = _transpose_call()(weight.T, w_tail)
    w_dense = w_packed.reshape(_V, D_MODEL)
    x2 = x.reshape(total // _SUB, _SUB).astype(jnp.int32)
    out = _gather_call(total)(w_dense, x2)
    return out.reshape(b, s, 128)[:, :, :D_MODEL]


# speedup vs baseline: 1.6717x; 1.6717x over previous
import functools

import jax
import jax.numpy as jnp
from jax import lax
from jax.experimental import pallas as pl
from jax.experimental.pallas import tpu as pltpu
from jax.experimental.pallas import tpu_sc as plsc

D_MODEL = 64
_INFO = plsc.get_sparse_core_info()
_NC, _NS = _INFO.num_cores, _INFO.num_subcores
_NW = _NC * _NS

_CHUNK = 512
_SUB = 128
_NSUB = _CHUNK // _SUB


def _emb_call(total_rows):
    n_per_w = total_rows // _NW
    n_chunks = n_per_w // _CHUNK
    n_pairs = n_chunks // 2
    mesh = plsc.VectorSubcoreMesh(core_axis_name="c", subcore_axis_name="s")

    @functools.partial(
        pl.kernel,
        out_type=jax.ShapeDtypeStruct((total_rows, 128), jnp.float32),
        mesh=mesh,
        scratch_types=[
            pltpu.VMEM((_NSUB, _SUB), jnp.int32),
            pltpu.VMEM((_NSUB, _SUB), jnp.int32),
            pltpu.VMEM((_CHUNK, D_MODEL), jnp.float32),
            pltpu.VMEM((_CHUNK, D_MODEL), jnp.float32),
            pltpu.SemaphoreType.DMA,
            pltpu.SemaphoreType.DMA,
            pltpu.SemaphoreType.DMA,
            pltpu.SemaphoreType.DMA,
            pltpu.SemaphoreType.DMA,
            pltpu.SemaphoreType.DMA,
        ],
        compiler_params=pltpu.CompilerParams(use_tc_tiling_on_sc=False),
    )
    def emb(w_hbm, x_hbm, out_hbm, idx0, idx1, rows0, rows1,
            isem0, isem1, gsem0, gsem1, osem0, osem1):
        wid = lax.axis_index("s") * _NC + lax.axis_index("c")
        base = wid * (n_per_w // _SUB)
        bufs = ((idx0, rows0, isem0, gsem0, osem0),
                (idx1, rows1, isem1, gsem1, osem1))

        for b in (0, 1):
            idxv, _, isem, _, _ = bufs[b]
            pltpu.async_copy(x_hbm.at[pl.ds(base + b * _NSUB, _NSUB)],
                             idxv, isem)

        def pair(p, carry):
            for b in (0, 1):
                idxv, rowsv, isem, gsem, osem = bufs[b]
                i = 2 * p + b
                g = base + i * _NSUB

                @pl.when(p > 0)
                def _():
                    pltpu.make_async_copy(
                        rowsv, out_hbm.at[pl.ds((g - 2 * _NSUB) * _SUB,
                                                _CHUNK),
                                          pl.ds(0, D_MODEL)], osem).wait()

                pltpu.make_async_copy(
                    x_hbm.at[pl.ds(g, _NSUB)], idxv, isem).wait()

                for j in range(_NSUB):
                    pltpu.async_copy(w_hbm.at[idxv.at[j]],
                                     rowsv.at[pl.ds(j * _SUB, _SUB)], gsem)

            for b in (0, 1):
                idxv, rowsv, isem, gsem, osem = bufs[b]
                i = 2 * p + b
                g = base + i * _NSUB

                for j in range(_NSUB):
                    pltpu.make_async_copy(
                        w_hbm.at[idxv.at[j]],
                        rowsv.at[pl.ds(j * _SUB, _SUB)], gsem).wait()

                g_next = base + lax.rem(i + 2, n_chunks) * _NSUB
                pltpu.async_copy(x_hbm.at[pl.ds(g_next, _NSUB)], idxv, isem)

                pltpu.async_copy(rowsv,
                                 out_hbm.at[pl.ds(g * _SUB, _CHUNK),
                                            pl.ds(0, D_MODEL)], osem)
            return carry

        lax.fori_loop(0, n_pairs, pair, 0)

        for b in (0, 1):
            idxv, rowsv, isem, _, osem = bufs[b]
            i = n_chunks - 2 + b
            g = base + i * _NSUB
            pltpu.make_async_copy(
                rowsv, out_hbm.at[pl.ds(g * _SUB, _CHUNK),
                                  pl.ds(0, D_MODEL)], osem).wait()
            pltpu.make_async_copy(
                x_hbm.at[pl.ds(base + b * _NSUB, _NSUB)], idxv, isem).wait()

    return emb


def kernel(x, weight):
    b, s = x.shape
    total = b * s
    x2 = x.reshape(total // _SUB, _SUB).astype(jnp.int32)
    out = _emb_call(total)(weight, x2)
    return out.reshape(b, s, 128)[:, :, :D_MODEL]


# speedup vs baseline: 1.6797x; 1.0048x over previous
import functools

import jax
import jax.numpy as jnp
from jax import lax
from jax.experimental import pallas as pl
from jax.experimental.pallas import tpu as pltpu
from jax.experimental.pallas import tpu_sc as plsc

D_MODEL = 64
_INFO = plsc.get_sparse_core_info()
_NC, _NS = _INFO.num_cores, _INFO.num_subcores
_NW = _NC * _NS

_CHUNK = 512
_SUB = 128
_NSUB = _CHUNK // _SUB


def _emb_call(total_rows):
    n_per_w = total_rows // _NW
    n_chunks = n_per_w // _CHUNK
    n_pairs = n_chunks // 2
    mesh = plsc.VectorSubcoreMesh(core_axis_name="c", subcore_axis_name="s")

    @functools.partial(
        pl.kernel,
        out_type=jax.ShapeDtypeStruct((total_rows, 128), jnp.float32),
        mesh=mesh,
        scratch_types=[
            pltpu.VMEM((_NSUB, _SUB), jnp.int32),
            pltpu.VMEM((_NSUB, _SUB), jnp.int32),
            pltpu.VMEM((_CHUNK, D_MODEL), jnp.float32),
            pltpu.VMEM((_CHUNK, D_MODEL), jnp.float32),
            pltpu.SemaphoreType.DMA,
            pltpu.SemaphoreType.DMA,
            pltpu.SemaphoreType.DMA,
            pltpu.SemaphoreType.DMA,
            pltpu.SemaphoreType.DMA,
            pltpu.SemaphoreType.DMA,
        ],
        compiler_params=pltpu.CompilerParams(use_tc_tiling_on_sc=False),
    )
    def emb(w_hbm, x_hbm, out_hbm, idx0, idx1, rows0, rows1,
            isem0, isem1, gsem0, gsem1, osem0, osem1):
        wid = lax.axis_index("s") * _NC + lax.axis_index("c")
        base = wid * (n_per_w // _SUB)
        bufs = ((idx0, rows0, isem0, gsem0, osem0),
                (idx1, rows1, isem1, gsem1, osem1))

        for b in (0, 1):
            idxv, _, isem, _, _ = bufs[b]
            pltpu.async_copy(x_hbm.at[pl.ds(base + b * _NSUB, _NSUB)],
                             idxv, isem)

        def pair(p, carry):
            for b in (0, 1):
                idxv, rowsv, isem, gsem, osem = bufs[b]
                i = 2 * p + b
                g = base + i * _NSUB

                @pl.when(p > 0)
                def _():
                    pltpu.make_async_copy(
                        rowsv, out_hbm.at[pl.ds((g - 2 * _NSUB) * _SUB,
                                                _CHUNK),
                                          pl.ds(0, D_MODEL)], osem).wait()

                pltpu.make_async_copy(
                    x_hbm.at[pl.ds(g, _NSUB)], idxv, isem).wait()

                for j in range(_NSUB):
                    pltpu.async_copy(w_hbm.at[idxv.at[j]],
                                     rowsv.at[pl.ds(j * _SUB, _SUB)], gsem)
                for j in range(_NSUB):
                    pltpu.make_async_copy(
                        w_hbm.at[idxv.at[j]],
                        rowsv.at[pl.ds(j * _SUB, _SUB)], gsem).wait()

                g_next = base + lax.rem(i + 2, n_chunks) * _NSUB
                pltpu.async_copy(x_hbm.at[pl.ds(g_next, _NSUB)], idxv, isem)

                pltpu.async_copy(rowsv,
                                 out_hbm.at[pl.ds(g * _SUB, _CHUNK),
                                            pl.ds(0, D_MODEL)], osem)
            return carry

        lax.fori_loop(0, n_pairs, pair, 0)

        for b in (0, 1):
            idxv, rowsv, isem, _, osem = bufs[b]
            i = n_chunks - 2 + b
            g = base + i * _NSUB
            pltpu.make_async_copy(
                rowsv, out_hbm.at[pl.ds(g * _SUB, _CHUNK),
                                  pl.ds(0, D_MODEL)], osem).wait()
            pltpu.make_async_copy(
                x_hbm.at[pl.ds(base + b * _NSUB, _NSUB)], idxv, isem).wait()

    return emb


def kernel(x, weight):
    b, s = x.shape
    total = b * s
    x2 = x.reshape(total // _SUB, _SUB).astype(jnp.int32)
    out = _emb_call(total)(weight, x2)
    return out.reshape(b, s, 128)[:, :, :D_MODEL]
